# Initial kernel scaffold; baseline (speedup 1.0000x reference)
import jax
import jax.numpy as jnp
from jax.experimental import pallas as pl


def kernel(x, hidden_states, masks, W_ih_l0, W_hh_l0, b_ih_l0, b_hh_l0, W_ih_l1, W_hh_l1, b_ih_l1, b_hh_l1):
    raise NotImplementedError("write your pallas kernel here")


# speedup vs baseline: 3.2074x; 3.2074x over previous
import functools

import jax
import jax.numpy as jnp
from jax.experimental import pallas as pl
from jax.experimental.pallas import tpu as pltpu


def _gru_layer_body(x_ref, m_ref, wih_ref, whh_ref, bih_ref, bhh_ref, h0_ref,
                    y_ref, hout_ref, h_s, gi_s, *, steps, n_envs, hid):
    i = pl.program_id(0)

    @pl.when(i == 0)
    def _():
        h_s[...] = h0_ref[...]

    gi_s[...] = jax.lax.dot_general(
        x_ref[...], wih_ref[...],
        (((1,), (1,)), ((), ())),
        preferred_element_type=jnp.float32,
    ) + bih_ref[...]

    whh = whh_ref[...]
    bhh = bhh_ref[...]

    def body(b, h):
        h = h * m_ref[b]
        gh = jax.lax.dot_general(
            h.astype(jnp.bfloat16), whh,
            (((1,), (1,)), ((), ())),
            preferred_element_type=jnp.float32,
        ) + bhh
        gi = gi_s[pl.ds(b * n_envs, n_envs), :]
        r = jax.nn.sigmoid(gi[:, :hid] + gh[:, :hid])
        z = jax.nn.sigmoid(gi[:, hid:2 * hid] + gh[:, hid:2 * hid])
        n = jnp.tanh(gi[:, 2 * hid:] + r * gh[:, 2 * hid:])
        h_new = (1.0 - z) * n + z * h
        y_ref[pl.ds(b * n_envs, n_envs), :] = h_new.astype(y_ref.dtype)
        return h_new

    h_fin = jax.lax.fori_loop(0, steps, body, h_s[...])
    h_s[...] = h_fin
    hout_ref[...] = h_fin


def _gru_layer(x, m3, wih, whh, bih, bhh, h0, out_dtype, block_t):
    n_envs, hid = h0.shape
    t = x.shape[0] // n_envs
    d = x.shape[1]
    grid = t // block_t
    bn = block_t * n_envs

    body = functools.partial(
        _gru_layer_body, steps=block_t, n_envs=n_envs, hid=hid)

    y, hout = pl.pallas_call(
        body,
        grid=(grid,),
        in_specs=[
            pl.BlockSpec((bn, d), lambda i: (i, 0)),
            pl.BlockSpec((block_t, n_envs, 1), lambda i: (i, 0, 0)),
            pl.BlockSpec((3 * hid, d), lambda i: (0, 0)),
            pl.BlockSpec((3 * hid, hid), lambda i: (0, 0)),
            pl.BlockSpec((1, 3 * hid), lambda i: (0, 0)),
            pl.BlockSpec((1, 3 * hid), lambda i: (0, 0)),
            pl.BlockSpec((n_envs, hid), lambda i: (0, 0)),
        ],
        out_specs=[
            pl.BlockSpec((bn, hid), lambda i: (i, 0)),
            pl.BlockSpec((n_envs, hid), lambda i: (0, 0)),
        ],
        out_shape=[
            jax.ShapeDtypeStruct((t * n_envs, hid), out_dtype),
            jax.ShapeDtypeStruct((n_envs, hid), jnp.float32),
        ],
        scratch_shapes=[
            pltpu.VMEM((n_envs, hid), jnp.float32),
            pltpu.VMEM((bn, 3 * hid), jnp.float32),
        ],
        compiler_params=pltpu.CompilerParams(
            dimension_semantics=("arbitrary",),
        ),
    )(x, m3, wih, whh, bih, bhh, h0)
    return y, hout


def kernel(x, hidden_states, masks, W_ih_l0, W_hh_l0, b_ih_l0, b_hh_l0,
           W_ih_l1, W_hh_l1, b_ih_l1, b_hh_l1):
    n_envs, n_layers, hid = hidden_states.shape
    t = x.shape[0] // n_envs

    block_t = 16
    while t % block_t:
        block_t //= 2

    m3 = masks.reshape(t, n_envs, 1)
    bf = jnp.bfloat16

    y0, h0f = _gru_layer(
        x.astype(bf), m3,
        W_ih_l0.astype(bf), W_hh_l0.astype(bf),
        b_ih_l0.reshape(1, -1), b_hh_l0.reshape(1, -1),
        hidden_states[:, 0, :], bf, block_t)
    y1, h1f = _gru_layer(
        y0, m3,
        W_ih_l1.astype(bf), W_hh_l1.astype(bf),
        b_ih_l1.reshape(1, -1), b_hh_l1.reshape(1, -1),
        hidden_states[:, 1, :], jnp.float32, block_t)

    hidden_out = jnp.stack([h0f, h1f], axis=1)
    return y1, hidden_out


# speedup vs baseline: 3.4961x; 1.0900x over previous
import functools

import jax
import jax.numpy as jnp
from jax.experimental import pallas as pl
from jax.experimental.pallas import tpu as pltpu


def _gru_layer_body(x_ref, m_ref, wih_ref, whh_ref, bih_ref, bhh_ref, h0_ref,
                    y_ref, hout_ref, h_s, gi_s, *, steps, n_envs, hid):
    i = pl.program_id(0)

    @pl.when(i == 0)
    def _():
        h_s[...] = h0_ref[...]

    gi_s[...] = jax.lax.dot_general(
        x_ref[...], wih_ref[...],
        (((1,), (1,)), ((), ())),
        preferred_element_type=jnp.float32,
    ) + bih_ref[...]

    whh = whh_ref[...]
    bhh = bhh_ref[...]

    h = h_s[...]
    for b in range(steps):
        h = h * m_ref[b]
        gh = jax.lax.dot_general(
            h.astype(jnp.bfloat16), whh,
            (((1,), (1,)), ((), ())),
            preferred_element_type=jnp.float32,
        ) + bhh
        gi = gi_s[b * n_envs:(b + 1) * n_envs, :]
        r = jax.nn.sigmoid(gi[:, :hid] + gh[:, :hid])
        z = jax.nn.sigmoid(gi[:, hid:2 * hid] + gh[:, hid:2 * hid])
        n = jnp.tanh(gi[:, 2 * hid:] + r * gh[:, 2 * hid:])
        h = (1.0 - z) * n + z * h
        y_ref[b * n_envs:(b + 1) * n_envs, :] = h.astype(y_ref.dtype)

    h_s[...] = h
    hout_ref[...] = h


def _gru_layer(x, m3, wih, whh, bih, bhh, h0, out_dtype, block_t):
    n_envs, hid = h0.shape
    t = x.shape[0] // n_envs
    d = x.shape[1]
    grid = t // block_t
    bn = block_t * n_envs

    body = functools.partial(
        _gru_layer_body, steps=block_t, n_envs=n_envs, hid=hid)

    y, hout = pl.pallas_call(
        body,
        grid=(grid,),
        in_specs=[
            pl.BlockSpec((bn, d), lambda i: (i, 0)),
            pl.BlockSpec((block_t, n_envs, 1), lambda i: (i, 0, 0)),
            pl.BlockSpec((3 * hid, d), lambda i: (0, 0)),
            pl.BlockSpec((3 * hid, hid), lambda i: (0, 0)),
            pl.BlockSpec((1, 3 * hid), lambda i: (0, 0)),
            pl.BlockSpec((1, 3 * hid), lambda i: (0, 0)),
            pl.BlockSpec((n_envs, hid), lambda i: (0, 0)),
        ],
        out_specs=[
            pl.BlockSpec((bn, hid), lambda i: (i, 0)),
            pl.BlockSpec((n_envs, hid), lambda i: (0, 0)),
        ],
        out_shape=[
            jax.ShapeDtypeStruct((t * n_envs, hid), out_dtype),
            jax.ShapeDtypeStruct((n_envs, hid), jnp.float32),
        ],
        scratch_shapes=[
            pltpu.VMEM((n_envs, hid), jnp.float32),
            pltpu.VMEM((bn, 3 * hid), jnp.float32),
        ],
        compiler_params=pltpu.CompilerParams(
            dimension_semantics=("arbitrary",),
        ),
    )(x, m3, wih, whh, bih, bhh, h0)
    return y, hout


def kernel(x, hidden_states, masks, W_ih_l0, W_hh_l0, b_ih_l0, b_hh_l0,
           W_ih_l1, W_hh_l1, b_ih_l1, b_hh_l1):
    n_envs, n_layers, hid = hidden_states.shape
    t = x.shape[0] // n_envs

    block_t = 16
    while t % block_t:
        block_t //= 2

    m3 = masks.reshape(t, n_envs, 1)
    bf = jnp.bfloat16

    y0, h0f = _gru_layer(
        x.astype(bf), m3,
        W_ih_l0.astype(bf), W_hh_l0.astype(bf),
        b_ih_l0.reshape(1, -1), b_hh_l0.reshape(1, -1),
        hidden_states[:, 0, :], bf, block_t)
    y1, h1f = _gru_layer(
        y0, m3,
        W_ih_l1.astype(bf), W_hh_l1.astype(bf),
        b_ih_l1.reshape(1, -1), b_hh_l1.reshape(1, -1),
        hidden_states[:, 1, :], jnp.float32, block_t)

    hidden_out = jnp.stack([h0f, h1f], axis=1)
    return y1, hidden_out
